# baseline (speedup 1.0000x reference)
import functools

import jax
import jax.numpy as jnp
from jax import lax
from jax.experimental import pallas as pl
from jax.experimental.pallas import tpu as pltpu
from jax.experimental.pallas import tpu_sc as plsc

VOCAB = 1000000
D = 64
B = 819200
NC = 2
NS = 16
NW = NC * NS
C = 128
NBLK = B // C
BLK_PER_W = NBLK // NW


def _sc_gather(idx2d, table):
    mesh = plsc.VectorSubcoreMesh(core_axis_name="c", subcore_axis_name="s")

    @functools.partial(
        pl.kernel,
        mesh=mesh,
        compiler_params=pltpu.CompilerParams(
            use_tc_tiling_on_sc=False, needs_layout_passes=False
        ),
        out_type=jax.ShapeDtypeStruct((D // 8, NBLK, 8 * C), jnp.float32),
        scratch_types=[
            pltpu.VMEM((BLK_PER_W, C), jnp.int32),
            [pltpu.VMEM((C, D), jnp.float32)] * 2,
            [pltpu.VMEM((D // 8, 8 * C), jnp.float32)] * 2,
            [pltpu.SemaphoreType.DMA] * 2,
            [pltpu.SemaphoreType.DMA] * 2,
        ],
    )
    def k(idx_hbm, table_hbm, out_hbm, idx_all, rows, outT, gsem, wsem):
        wid = lax.axis_index("s") * NC + lax.axis_index("c")
        blk0 = wid * BLK_PER_W
        pltpu.sync_copy(idx_hbm.at[pl.ds(blk0, BLK_PER_W)], idx_all)

        lane = lax.iota(jnp.int32, 16)
        tok_idx = [lane + 16 * g for g in range(C // 16)]
        zero16 = lane - lane

        def fire_gather(t, b):
            pltpu.async_copy(table_hbm.at[idx_all.at[t]], rows[b], gsem[b])

        def drain_gather(b):
            pltpu.make_async_copy(
                table_hbm.at[pl.ds(0, C)], rows[b], gsem[b]
            ).wait()

        def fire_write(t, b):
            pltpu.async_copy(outT[b], out_hbm.at[:, blk0 + t], wsem[b])

        def drain_write(b):
            pltpu.make_async_copy(outT[b], out_hbm.at[:, 0], wsem[b]).wait()

        def transpose_block(b):
            def dbody(d8, carry):
                for u in range(8):
                    dvec = zero16 + (d8 * 8 + u)
                    for g in range(C // 16):
                        v = plsc.load_gather(rows[b], [tok_idx[g], dvec])
                        outT[b][d8, pl.ds(u * C + 16 * g, 16)] = v
                return carry

            lax.fori_loop(0, D // 8, dbody, 0)

        fire_gather(0, 0)
        fire_gather(1, 1)

        drain_gather(0)
        transpose_block(0)
        fire_write(0, 0)
        fire_gather(2, 0)
        drain_gather(1)
        transpose_block(1)
        fire_write(1, 1)
        fire_gather(3, 1)

        def body(i, carry):
            for u in range(2):
                t = 2 * i + u
                b = u
                drain_gather(b)
                drain_write(b)
                transpose_block(b)
                fire_write(t, b)
                fire_gather(t + 2, b)
            return carry

        lax.fori_loop(1, BLK_PER_W // 2 - 1, body, 0)

        drain_gather(0)
        drain_write(0)
        transpose_block(0)
        fire_write(BLK_PER_W - 2, 0)
        drain_gather(1)
        drain_write(1)
        transpose_block(1)
        fire_write(BLK_PER_W - 1, 1)
        drain_write(0)
        drain_write(1)

    return k(idx2d, table)


def kernel(indices, table):
    idx2d = indices.astype(jnp.int32).reshape(NBLK, C)
    o3 = _sc_gather(idx2d, table)
    o4 = jnp.reshape(o3, (D // 8, NBLK, 8, C))
    return jnp.transpose(o4, (1, 3, 0, 2)).reshape(B, D)
